# Initial kernel scaffold; baseline (speedup 1.0000x reference)
import jax
import jax.numpy as jnp
from jax.experimental import pallas as pl


def kernel(embeddings, edges_ij, edge_features, edge_status, W3, W4):
    raise NotImplementedError("write your pallas kernel here")


# speedup vs baseline: 54.8704x; 54.8704x over previous
import functools

import jax
import jax.numpy as jnp
from jax import lax
from jax.experimental import pallas as pl
from jax.experimental.pallas import tpu as pltpu
from jax.experimental.pallas import tpu_sc as plsc

B, N, D, E, Fe = 2, 10000, 128, 320000, 16

NS = 16
EP = E // NS
K = 80
C = EP // K
G = 25
NG = C // G
RT = 80
NT = N // RT


def _s1_body(ef_ref, g_ref, w4_ref, out_ref):
    ef = ef_ref[0]
    w4 = w4_ref[...]
    x = lax.dot_general(ef, w4, (((1,), (1,)), ((), ())),
                        preferred_element_type=jnp.float32)
    x = jnp.where(x >= 0, x, 0.01 * x)
    out_ref[0] = x * g_ref[0]


def _stage1(edge_features, gate_col, W4):
    T = 2560
    grid = (B, E // T)
    return pl.pallas_call(
        _s1_body,
        grid=grid,
        in_specs=[
            pl.BlockSpec((1, T, Fe), lambda b, i: (b, i, 0)),
            pl.BlockSpec((1, T, 1), lambda b, i: (b, i, 0)),
            pl.BlockSpec((D, Fe), lambda b, i: (0, 0)),
        ],
        out_specs=pl.BlockSpec((1, T, D), lambda b, i: (b, i, 0)),
        out_shape=jax.ShapeDtypeStruct((B, E, D), jnp.float32),
    )(edge_features, gate_col, W4)


def _sc_body(x4_hbm, u_hbm, v_hbm, out_hbm, u_buf, v_buf, rows, stage, msg_sh):
    c = lax.axis_index("c")
    s = lax.axis_index("s")

    def _zrow(i, _):
        def _zcol(j, _):
            stage[i, pl.ds(j * 16, 16)] = jnp.zeros((16,), jnp.float32)
            return 0
        return lax.fori_loop(0, D // 16, _zcol, 0)
    lax.fori_loop(0, RT, _zrow, 0)
    for j in range(pl.cdiv(NT, NS)):
        t = s + j * NS

        @pl.when(t < NT)
        def _():
            pltpu.sync_copy(stage, msg_sh.at[pl.ds(t * RT, RT)])
    plsc.subcore_barrier()

    def _group(gi, _):
        pltpu.sync_copy(u_hbm.at[s, gi], u_buf)
        pltpu.sync_copy(v_hbm.at[s, gi], v_buf)

        def _chunk(jj, _):
            base = s * EP + (gi * G + jj) * K
            pltpu.sync_copy(x4_hbm.at[c, pl.ds(base, K)], rows)
            pltpu.sync_copy(rows, msg_sh.at[u_buf.at[jj]], add=True)
            pltpu.sync_copy(rows, msg_sh.at[v_buf.at[jj]], add=True)
            return 0
        return lax.fori_loop(0, G, _chunk, 0)
    lax.fori_loop(0, NG, _group, 0)
    plsc.subcore_barrier()

    for j in range(pl.cdiv(NT, NS)):
        t = s + j * NS

        @pl.when(t < NT)
        def _():
            pltpu.sync_copy(msg_sh.at[pl.ds(t * RT, RT)], stage)
            pltpu.sync_copy(stage, out_hbm.at[c, pl.ds(t * RT, RT)])


@functools.lru_cache(maxsize=1)
def _sc_scatter():
    return pl.kernel(
        _sc_body,
        out_type=jax.ShapeDtypeStruct((B, N, D), jnp.float32),
        mesh=plsc.VectorSubcoreMesh(core_axis_name="c", subcore_axis_name="s"),
        scratch_types=[
            pltpu.VMEM((G, K), jnp.int32),
            pltpu.VMEM((G, K), jnp.int32),
            pltpu.VMEM((K, D), jnp.float32),
            pltpu.VMEM((RT, D), jnp.float32),
            pltpu.VMEM_SHARED((N, D), jnp.float32),
        ],
    )


def _s3_body(msg_ref, w3_ref, out_ref):
    out_ref[0] = lax.dot_general(msg_ref[0], w3_ref[...],
                                 (((1,), (1,)), ((), ())),
                                 preferred_element_type=jnp.float32)


def _stage3(msg, W3):
    return pl.pallas_call(
        _s3_body,
        grid=(B,),
        in_specs=[
            pl.BlockSpec((1, N, D), lambda b: (b, 0, 0)),
            pl.BlockSpec((D, D), lambda b: (0, 0)),
        ],
        out_specs=pl.BlockSpec((1, N, D), lambda b: (b, 0, 0)),
        out_shape=jax.ShapeDtypeStruct((B, N, D), jnp.float32),
    )(msg, W3)


def kernel(embeddings, edges_ij, edge_features, edge_status, W3, W4):
    g = edge_status
    if g.ndim == 1:
        g = g[None, :]
    g = jnp.broadcast_to(g, (B, E)).astype(jnp.float32)
    gate_col = g[:, :, None]

    x4 = _stage1(edge_features, gate_col, W4)

    u2 = edges_ij[:, 0].reshape(NS, NG, G, K)
    v2 = edges_ij[:, 1].reshape(NS, NG, G, K)
    msg = _sc_scatter()(x4, u2, v2)

    return _stage3(msg, W3)
